# baseline (speedup 1.0000x reference)
import functools

import jax
import jax.numpy as jnp
from jax import lax
from jax.experimental import pallas as pl
from jax.experimental.pallas import tpu as pltpu
from jax.experimental.pallas import tpu_sc as plsc

_B = 16384
_D = 64
_DP = 128
_L = 16

_info = plsc.get_sparse_core_info()
_NC, _NS = _info.num_cores, _info.num_subcores
_NW = _NC * _NS
_BPW = _B // _NW
_CH = 128
_NCH = _BPW // _CH

_mesh = plsc.VectorSubcoreMesh(core_axis_name="c", subcore_axis_name="s")


@functools.partial(
    pl.kernel,
    out_type=[
        jax.ShapeDtypeStruct((_B,), jnp.float32),
        jax.ShapeDtypeStruct((_NW, _L), jnp.float32),
    ],
    mesh=_mesh,
    scratch_types=[
        pltpu.VMEM((_BPW,), jnp.int32),
        pltpu.VMEM((_BPW,), jnp.int32),
        pltpu.VMEM((2, _CH, _DP), jnp.float32),
        pltpu.VMEM((2, _CH, _DP), jnp.float32),
        pltpu.VMEM((_BPW,), jnp.float32),
        pltpu.VMEM((_L,), jnp.float32),
        pltpu.SemaphoreType.DMA,
        pltpu.SemaphoreType.DMA,
        pltpu.SemaphoreType.DMA,
        pltpu.SemaphoreType.DMA,
    ],
)
def _mf_kernel(uidx_hbm, iidx_hbm, utab_hbm, itab_hbm, pred_hbm, partials_hbm,
               uidx_v, iidx_v, ubuf, ibuf, pred_v, accsq_v,
               sem_u0, sem_u1, sem_i0, sem_i1):
    wid = lax.axis_index("s") * _NC + lax.axis_index("c")
    base = wid * _BPW

    pltpu.sync_copy(uidx_hbm.at[pl.ds(base, _BPW)], uidx_v)
    pltpu.sync_copy(iidx_hbm.at[pl.ds(base, _BPW)], iidx_v)

    sems_u = (sem_u0, sem_u1)
    sems_i = (sem_i0, sem_i1)
    lane = lax.iota(jnp.int32, _L)

    def issue(c, slot):
        off = c * _CH
        pltpu.async_copy(utab_hbm.at[uidx_v.at[pl.ds(off, _CH)]],
                         ubuf.at[slot], sems_u[slot])
        pltpu.async_copy(itab_hbm.at[iidx_v.at[pl.ds(off, _CH)]],
                         ibuf.at[slot], sems_i[slot])

    def wait(slot):
        pltpu.make_async_copy(utab_hbm.at[uidx_v.at[pl.ds(0, _CH)]],
                              ubuf.at[slot], sems_u[slot]).wait()
        pltpu.make_async_copy(itab_hbm.at[iidx_v.at[pl.ds(0, _CH)]],
                              ibuf.at[slot], sems_i[slot]).wait()

    issue(0, 0)
    issue(1, 1)

    accsq = jnp.zeros((_L,), jnp.float32)
    for c in range(_NCH):
        slot = c & 1
        wait(slot)

        def group_body(g, accsq, c=c, slot=slot):
            base_r = pl.multiple_of(g * _L, _L)
            preds = jnp.zeros((_L,), jnp.float32)
            for r in range(_L):
                prod = jnp.zeros((_L,), jnp.float32)
                for k in range(_D // _L):
                    u = ubuf[slot, base_r + r, pl.ds(k * _L, _L)]
                    i = ibuf[slot, base_r + r, pl.ds(k * _L, _L)]
                    prod = prod + u * i
                    accsq = accsq + (u * u + i * i)
                for sh in (8, 4, 2, 1):
                    prod = prod + prod.at[lane ^ sh].get(
                        mode="promise_in_bounds")
                preds = jnp.where(lane == r, prod, preds)
            pred_v[pl.ds(c * _CH + base_r, _L)] = preds
            return accsq

        accsq = lax.fori_loop(0, _CH // _L, group_body, accsq)
        if c + 2 < _NCH:
            issue(c + 2, slot)
    accsq_v[...] = accsq

    pltpu.sync_copy(pred_v, pred_hbm.at[pl.ds(base, _BPW)])
    pltpu.sync_copy(accsq_v, partials_hbm.at[wid])


def kernel(user_indices, item_indices, user_embedding_weight, item_embedding_weight):
    utab_p = jnp.pad(user_embedding_weight, ((0, 0), (0, _DP - _D)))
    itab_p = jnp.pad(item_embedding_weight, ((0, 0), (0, _DP - _D)))
    pred, partials = _mf_kernel(
        user_indices.astype(jnp.int32),
        item_indices.astype(jnp.int32),
        utab_p,
        itab_p,
    )
    reg_loss = 0.5 * jnp.sum(partials) / float(_B)
    return pred, reg_loss
